# baseline (speedup 1.0000x reference)
import functools
import math

import jax
import jax.numpy as jnp
from jax import lax
from jax.experimental import pallas as pl
from jax.experimental.pallas import tpu as pltpu
from jax.experimental.pallas import tpu_sc as plsc

NC = 2
NS = 16
NW = NC * NS
L = 16

B, S, K = 16, 8192, 32
R = B * S
SPLIT = 2
R_SC = SPLIT * S
ROWS_PER_W = R_SC // NW
NCHUNK = 2
CHUNK = ROWS_PER_W // NCHUNK
LOG_EPS = float(math.log(1e-30))
LN2 = 0.6931471805599453
SQRT2 = 1.4142135623730951
TCB = 128
GEXP = 32
SBLK = 4096


def _fast_log(x):
    bits = lax.bitcast_convert_type(x, jnp.int32)
    e = lax.shift_right_logical(bits, 23) - 127
    m = lax.bitcast_convert_type(
        jnp.bitwise_or(jnp.bitwise_and(bits, 0x7FFFFF), 0x3F800000), jnp.float32)
    big = m >= SQRT2
    m = jnp.where(big, m * 0.5, m)
    e = jnp.where(big, e + 1, e).astype(jnp.float32)
    s = (m - 1.0) / (m + 1.0)
    z = s * s
    poly = 1.0 + z * (1.0 / 3.0 + z * (1.0 / 5.0 + z * (1.0 / 7.0 + z * (1.0 / 9.0))))
    return e * LN2 + 2.0 * s * poly


def _sc_body(deg_hbm, qv_hbm, j_hbm, lh_hbm, lp_hbm, qv_v,
             d_v0, d_v1, j_v0, j_v1, lh_v0, lh_v1, lp_v0, lp_v1,
             sem_in, sem_out):
    wid = lax.axis_index("s") * NC + lax.axis_index("c")
    base = wid * ROWS_PER_W

    d_bufs = (d_v0, d_v1)
    j_bufs = (j_v0, j_v1)
    lh_bufs = (lh_v0, lh_v1)
    lp_bufs = (lp_v0, lp_v1)

    pltpu.sync_copy(qv_hbm, qv_v)

    def in_copy(c, buf):
        return pltpu.make_async_copy(
            deg_hbm.at[pl.ds(base + c * CHUNK, CHUNK)], d_bufs[buf], sem_in.at[buf])

    def out_copy(c, buf):
        sl = pl.ds(base + c * CHUNK, CHUNK)
        return (pltpu.make_async_copy(j_bufs[buf], j_hbm.at[sl], sem_out.at[buf]),
                pltpu.make_async_copy(lh_bufs[buf], lh_hbm.at[sl], sem_out.at[buf]),
                pltpu.make_async_copy(lp_bufs[buf], lp_hbm.at[sl], sem_out.at[buf]))

    in_copy(0, 0).start()

    i31 = jnp.full((L,), K - 1, jnp.int32)

    for c in range(NCHUNK):
        buf = c % 2
        if c + 1 < NCHUNK:
            in_copy(c + 1, 1 - buf).start()
        in_copy(c, buf).wait()

        qmax = plsc.load_gather(qv_v, [i31])
        d_v = d_bufs[buf]
        j_v = j_bufs[buf]
        lh_v = lh_bufs[buf]
        lp_v = lp_bufs[buf]

        def step(i, _):
            d = d_v[pl.ds(i * L, L)]
            j = jnp.zeros((L,), jnp.int32)
            for stepw in (16, 8, 4, 2, 1):
                cand = j + stepw
                v = plsc.load_gather(qv_v, [jnp.minimum(cand, K - 1)])
                j = jnp.where((cand <= K - 1) & (d >= v), cand, j)
            lower = plsc.load_gather(qv_v, [j])
            upper = plsc.load_gather(qv_v, [jnp.minimum(j + 1, K - 1)])
            pos = (d - lower) / (upper - lower + 1e-10)
            pos = jnp.clip(pos, 0.0, 1.0)
            m = (d >= lower) & (d < upper)
            over = d >= qmax
            jenc = jnp.where(over, K, jnp.where(m, j, -1)).astype(jnp.float32)
            sl = pl.ds(i * L, L)
            j_v[sl] = jenc
            lh_v[sl] = _fast_log(1.0 - pos + 1e-30)
            lp_v[sl] = _fast_log(pos + 1e-30)
            return 0

        lax.fori_loop(0, CHUNK // L, step, 0, unroll=2)
        for cp in out_copy(c, buf):
            cp.start()

    for cc in range(NCHUNK):
        for cp in out_copy(cc, cc % 2):
            cp.wait()


def _expand_store(jbr, lhr, lpr, o_ref):
    gsel = lax.broadcasted_iota(jnp.int32, (GEXP, GEXP * K), 1) // K
    grow = lax.broadcasted_iota(jnp.int32, (GEXP, GEXP * K), 0)
    ee = (gsel == grow).astype(jnp.float32)
    dn = (((0,), (0,)), ((), ()))
    jb = lax.dot_general(jbr, ee, dn, preferred_element_type=jnp.float32)
    lb = lax.dot_general(lhr, ee, dn, preferred_element_type=jnp.float32)
    pb = lax.dot_general(lpr, ee, dn, preferred_element_type=jnp.float32)
    col = (lax.broadcasted_iota(jnp.int32, (TCB, GEXP * K), 1) %
           K).astype(jnp.float32)
    out = jnp.where(col == jb, lb, LOG_EPS)
    v31 = jnp.where(jb == float(K), 0.0,
                    jnp.where(jb == float(K - 2), pb, LOG_EPS))
    out = jnp.where(col == float(K - 1), v31, out)
    for g in range(GEXP):
        o_ref[0, g * TCB:(g + 1) * TCB, :] = lax.slice(
            out, (0, g * K), (TCB, g * K + K))


def _tc_merged_body(d_ref, qv_ref, qvn_ref, j_ref, lh_ref, lp_ref, o_ref):
    bidx = pl.program_id(0)

    @pl.when(bidx < SPLIT)
    def _():
        _expand_store(j_ref[...], lh_ref[...], lp_ref[...], o_ref)

    @pl.when(bidx >= SPLIT)
    def _():
        d2 = d_ref[...]
        j = jnp.zeros(d2.shape, jnp.int32)
        lower = jnp.full(d2.shape, qv_ref[0], jnp.float32)
        upper = jnp.full(d2.shape, qvn_ref[0], jnp.float32)
        for c in range(K):
            gec = d2 >= qv_ref[c]
            j = j + gec.astype(jnp.int32)
            lower = jnp.where(gec, qv_ref[c], lower)
            upper = jnp.where(gec, qvn_ref[c], upper)
        pos = (d2 - lower) / (upper - lower + 1e-10)
        pos = jnp.clip(pos, 0.0, 1.0)
        m = (d2 >= lower) & (d2 < upper)
        over = d2 >= qv_ref[K - 1]
        jenc = jnp.where(over, K, jnp.where(m, j - 1, -1)).astype(jnp.float32)
        lh = _fast_log(1.0 - pos + 1e-30)
        lp = _fast_log(pos + 1e-30)
        _expand_store(jenc, lh, lp, o_ref)


@jax.jit
def kernel(degrees, quantile_values):
    qv = quantile_values
    qvn = jnp.concatenate([qv[1:], qv[K - 1:]])
    deg_sc = degrees.reshape(R)[:R_SC]

    mesh = plsc.VectorSubcoreMesh(
        core_axis_name="c", subcore_axis_name="s", num_cores=NC, num_subcores=NS)
    j_arr, lh_arr, lp_arr = pl.kernel(
        _sc_body,
        out_type=(jax.ShapeDtypeStruct((R_SC,), jnp.float32),
                  jax.ShapeDtypeStruct((R_SC,), jnp.float32),
                  jax.ShapeDtypeStruct((R_SC,), jnp.float32)),
        mesh=mesh,
        compiler_params=pltpu.CompilerParams(needs_layout_passes=False),
        scratch_types=[
            pltpu.VMEM((K,), jnp.float32),
            pltpu.VMEM((CHUNK,), jnp.float32),
            pltpu.VMEM((CHUNK,), jnp.float32),
            pltpu.VMEM((CHUNK,), jnp.float32),
            pltpu.VMEM((CHUNK,), jnp.float32),
            pltpu.VMEM((CHUNK,), jnp.float32),
            pltpu.VMEM((CHUNK,), jnp.float32),
            pltpu.VMEM((CHUNK,), jnp.float32),
            pltpu.VMEM((CHUNK,), jnp.float32),
            pltpu.SemaphoreType.DMA((2,)),
            pltpu.SemaphoreType.DMA((2,)),
        ],
    )(deg_sc, qv)

    nsb = S // SBLK
    deg2 = degrees.reshape(R // TCB, TCB)
    j2 = j_arr.reshape(R_SC // TCB, TCB)
    lh2 = lh_arr.reshape(R_SC // TCB, TCB)
    lp2 = lp_arr.reshape(R_SC // TCB, TCB)
    nsc = R_SC // TCB // GEXP

    def scal_map(b, s):
        i = b * nsb + s
        return (jnp.minimum(i, nsc - 1), 0)

    out = pl.pallas_call(
        _tc_merged_body,
        grid=(B, nsb),
        in_specs=[
            pl.BlockSpec((GEXP, TCB), lambda b, s: (b * nsb + s, 0)),
            pl.BlockSpec(memory_space=pltpu.MemorySpace.SMEM),
            pl.BlockSpec(memory_space=pltpu.MemorySpace.SMEM),
            pl.BlockSpec((GEXP, TCB), scal_map),
            pl.BlockSpec((GEXP, TCB), scal_map),
            pl.BlockSpec((GEXP, TCB), scal_map),
        ],
        out_specs=pl.BlockSpec((1, SBLK, K), lambda b, s: (b, s, 0)),
        out_shape=jax.ShapeDtypeStruct((B, S, K), jnp.float32),
    )(deg2, qv, qvn, j2, lh2, lp2)
    return out


# speedup vs baseline: 1.0979x; 1.0979x over previous
import functools
import math

import jax
import jax.numpy as jnp
from jax import lax
from jax.experimental import pallas as pl
from jax.experimental.pallas import tpu as pltpu
from jax.experimental.pallas import tpu_sc as plsc

NC = 2
NS = 16
NW = NC * NS
L = 16

B, S, K = 16, 8192, 32
R = B * S
SPLIT = 2
R_SC = SPLIT * S
ROWS_PER_W = R_SC // NW
NCHUNK = 2
CHUNK = ROWS_PER_W // NCHUNK
LOG_EPS = float(math.log(1e-30))
LN2 = 0.6931471805599453
SQRT2 = 1.4142135623730951
TCB = 128
GEXP = 64
SBLK = 8192


def _fast_log(x):
    bits = lax.bitcast_convert_type(x, jnp.int32)
    e = lax.shift_right_logical(bits, 23) - 127
    m = lax.bitcast_convert_type(
        jnp.bitwise_or(jnp.bitwise_and(bits, 0x7FFFFF), 0x3F800000), jnp.float32)
    big = m >= SQRT2
    m = jnp.where(big, m * 0.5, m)
    e = jnp.where(big, e + 1, e).astype(jnp.float32)
    s = (m - 1.0) / (m + 1.0)
    z = s * s
    poly = 1.0 + z * (1.0 / 3.0 + z * (1.0 / 5.0 + z * (1.0 / 7.0 + z * (1.0 / 9.0))))
    return e * LN2 + 2.0 * s * poly


def _sc_body(deg_hbm, qv_hbm, j_hbm, lh_hbm, lp_hbm, qv_v,
             d_v0, d_v1, j_v0, j_v1, lh_v0, lh_v1, lp_v0, lp_v1,
             sem_in, sem_out):
    wid = lax.axis_index("s") * NC + lax.axis_index("c")
    base = wid * ROWS_PER_W

    d_bufs = (d_v0, d_v1)
    j_bufs = (j_v0, j_v1)
    lh_bufs = (lh_v0, lh_v1)
    lp_bufs = (lp_v0, lp_v1)

    pltpu.sync_copy(qv_hbm, qv_v)

    def in_copy(c, buf):
        return pltpu.make_async_copy(
            deg_hbm.at[pl.ds(base + c * CHUNK, CHUNK)], d_bufs[buf], sem_in.at[buf])

    def out_copy(c, buf):
        sl = pl.ds(base + c * CHUNK, CHUNK)
        return (pltpu.make_async_copy(j_bufs[buf], j_hbm.at[sl], sem_out.at[buf]),
                pltpu.make_async_copy(lh_bufs[buf], lh_hbm.at[sl], sem_out.at[buf]),
                pltpu.make_async_copy(lp_bufs[buf], lp_hbm.at[sl], sem_out.at[buf]))

    in_copy(0, 0).start()

    i31 = jnp.full((L,), K - 1, jnp.int32)

    for c in range(NCHUNK):
        buf = c % 2
        if c + 1 < NCHUNK:
            in_copy(c + 1, 1 - buf).start()
        in_copy(c, buf).wait()

        qmax = plsc.load_gather(qv_v, [i31])
        d_v = d_bufs[buf]
        j_v = j_bufs[buf]
        lh_v = lh_bufs[buf]
        lp_v = lp_bufs[buf]

        def step(i, _):
            d = d_v[pl.ds(i * L, L)]
            j = jnp.zeros((L,), jnp.int32)
            for stepw in (16, 8, 4, 2, 1):
                cand = j + stepw
                v = plsc.load_gather(qv_v, [jnp.minimum(cand, K - 1)])
                j = jnp.where((cand <= K - 1) & (d >= v), cand, j)
            lower = plsc.load_gather(qv_v, [j])
            upper = plsc.load_gather(qv_v, [jnp.minimum(j + 1, K - 1)])
            pos = (d - lower) / (upper - lower + 1e-10)
            pos = jnp.clip(pos, 0.0, 1.0)
            m = (d >= lower) & (d < upper)
            over = d >= qmax
            jenc = jnp.where(over, K, jnp.where(m, j, -1)).astype(jnp.float32)
            sl = pl.ds(i * L, L)
            j_v[sl] = jenc
            lh_v[sl] = _fast_log(1.0 - pos + 1e-30)
            lp_v[sl] = _fast_log(pos + 1e-30)
            return 0

        lax.fori_loop(0, CHUNK // L, step, 0, unroll=2)
        for cp in out_copy(c, buf):
            cp.start()

    for cc in range(NCHUNK):
        for cp in out_copy(cc, cc % 2):
            cp.wait()


def _expand_store(jbr, lhr, lpr, o_ref):
    gsel = lax.broadcasted_iota(jnp.int32, (GEXP, GEXP * K), 1) // K
    grow = lax.broadcasted_iota(jnp.int32, (GEXP, GEXP * K), 0)
    ee = (gsel == grow).astype(jnp.float32)
    dn = (((0,), (0,)), ((), ()))
    jb = lax.dot_general(jbr, ee, dn, preferred_element_type=jnp.float32)
    lb = lax.dot_general(lhr, ee, dn, preferred_element_type=jnp.float32)
    pb = lax.dot_general(lpr, ee, dn, preferred_element_type=jnp.float32)
    col = (lax.broadcasted_iota(jnp.int32, (TCB, GEXP * K), 1) %
           K).astype(jnp.float32)
    out = jnp.where(col == jb, lb, LOG_EPS)
    v31 = jnp.where(jb == float(K), 0.0,
                    jnp.where(jb == float(K - 2), pb, LOG_EPS))
    out = jnp.where(col == float(K - 1), v31, out)
    for g in range(GEXP):
        o_ref[0, g * TCB:(g + 1) * TCB, :] = lax.slice(
            out, (0, g * K), (TCB, g * K + K))


def _tc_compute_body(d_ref, qv_ref, qvn_ref, o_ref):
    d2 = d_ref[...]
    j = jnp.zeros(d2.shape, jnp.int32)
    lower = jnp.full(d2.shape, qv_ref[0], jnp.float32)
    upper = jnp.full(d2.shape, qvn_ref[0], jnp.float32)
    for c in range(K):
        gec = d2 >= qv_ref[c]
        j = j + gec.astype(jnp.int32)
        lower = jnp.where(gec, qv_ref[c], lower)
        upper = jnp.where(gec, qvn_ref[c], upper)
    pos = (d2 - lower) / (upper - lower + 1e-10)
    pos = jnp.clip(pos, 0.0, 1.0)
    m = (d2 >= lower) & (d2 < upper)
    over = d2 >= qv_ref[K - 1]
    jenc = jnp.where(over, K, jnp.where(m, j - 1, -1)).astype(jnp.float32)
    lh = _fast_log(1.0 - pos + 1e-30)
    lp = _fast_log(pos + 1e-30)
    _expand_store(jenc, lh, lp, o_ref)


def _tc_expand_body(j_ref, lh_ref, lp_ref, prev_ref, o_ref):
    del prev_ref
    _expand_store(j_ref[...], lh_ref[...], lp_ref[...], o_ref)


@jax.jit
def kernel(degrees, quantile_values):
    qv = quantile_values
    qvn = jnp.concatenate([qv[1:], qv[K - 1:]])
    deg_sc = degrees.reshape(R)[:R_SC]

    nsb = S // SBLK
    deg2 = degrees.reshape(R // TCB, TCB)
    out1 = pl.pallas_call(
        _tc_compute_body,
        grid=(B - SPLIT, nsb),
        in_specs=[
            pl.BlockSpec((GEXP, TCB), lambda b, s: ((b + SPLIT) * nsb + s, 0)),
            pl.BlockSpec(memory_space=pltpu.MemorySpace.SMEM),
            pl.BlockSpec(memory_space=pltpu.MemorySpace.SMEM),
        ],
        out_specs=pl.BlockSpec((1, SBLK, K), lambda b, s: (b + SPLIT, s, 0)),
        out_shape=jax.ShapeDtypeStruct((B, S, K), jnp.float32),
    )(deg2, qv, qvn)

    mesh = plsc.VectorSubcoreMesh(
        core_axis_name="c", subcore_axis_name="s", num_cores=NC, num_subcores=NS)
    j_arr, lh_arr, lp_arr = pl.kernel(
        _sc_body,
        out_type=(jax.ShapeDtypeStruct((R_SC,), jnp.float32),
                  jax.ShapeDtypeStruct((R_SC,), jnp.float32),
                  jax.ShapeDtypeStruct((R_SC,), jnp.float32)),
        mesh=mesh,
        compiler_params=pltpu.CompilerParams(needs_layout_passes=False),
        scratch_types=[
            pltpu.VMEM((K,), jnp.float32),
            pltpu.VMEM((CHUNK,), jnp.float32),
            pltpu.VMEM((CHUNK,), jnp.float32),
            pltpu.VMEM((CHUNK,), jnp.float32),
            pltpu.VMEM((CHUNK,), jnp.float32),
            pltpu.VMEM((CHUNK,), jnp.float32),
            pltpu.VMEM((CHUNK,), jnp.float32),
            pltpu.VMEM((CHUNK,), jnp.float32),
            pltpu.VMEM((CHUNK,), jnp.float32),
            pltpu.SemaphoreType.DMA((2,)),
            pltpu.SemaphoreType.DMA((2,)),
        ],
    )(deg_sc, qv)

    j2 = j_arr.reshape(R_SC // TCB, TCB)
    lh2 = lh_arr.reshape(R_SC // TCB, TCB)
    lp2 = lp_arr.reshape(R_SC // TCB, TCB)
    out = pl.pallas_call(
        _tc_expand_body,
        grid=(SPLIT, nsb),
        in_specs=[
            pl.BlockSpec((GEXP, TCB), lambda b, s: (b * nsb + s, 0)),
            pl.BlockSpec((GEXP, TCB), lambda b, s: (b * nsb + s, 0)),
            pl.BlockSpec((GEXP, TCB), lambda b, s: (b * nsb + s, 0)),
            pl.BlockSpec(memory_space=pltpu.MemorySpace.HBM),
        ],
        out_specs=pl.BlockSpec((1, SBLK, K), lambda b, s: (b, s, 0)),
        out_shape=jax.ShapeDtypeStruct((B, S, K), jnp.float32),
        input_output_aliases={3: 0},
    )(j2, lh2, lp2, out1)
    return out


# speedup vs baseline: 1.1081x; 1.0093x over previous
import functools
import math

import jax
import jax.numpy as jnp
from jax import lax
from jax.experimental import pallas as pl
from jax.experimental.pallas import tpu as pltpu
from jax.experimental.pallas import tpu_sc as plsc

NC = 2
NS = 16
NW = NC * NS
L = 16

B, S, K = 16, 8192, 32
R = B * S
SPLIT = 2
R_SC = SPLIT * S
ROWS_PER_W = R_SC // NW
NCHUNK = 2
CHUNK = ROWS_PER_W // NCHUNK
LOG_EPS = float(math.log(1e-30))
LN2 = 0.6931471805599453
SQRT2 = 1.4142135623730951
TCB = 128
GEXP = 128
SBLK = 8192


def _fast_log(x):
    bits = lax.bitcast_convert_type(x, jnp.int32)
    e = lax.shift_right_logical(bits, 23) - 127
    m = lax.bitcast_convert_type(
        jnp.bitwise_or(jnp.bitwise_and(bits, 0x7FFFFF), 0x3F800000), jnp.float32)
    big = m >= SQRT2
    m = jnp.where(big, m * 0.5, m)
    e = jnp.where(big, e + 1, e).astype(jnp.float32)
    s = (m - 1.0) / (m + 1.0)
    z = s * s
    poly = 1.0 + z * (1.0 / 3.0 + z * (1.0 / 5.0 + z * (1.0 / 7.0 + z * (1.0 / 9.0))))
    return e * LN2 + 2.0 * s * poly


def _sc_body(deg_hbm, qv_hbm, j_hbm, lh_hbm, lp_hbm, qv_v,
             d_v0, d_v1, j_v0, j_v1, lh_v0, lh_v1, lp_v0, lp_v1,
             sem_in, sem_out):
    wid = lax.axis_index("s") * NC + lax.axis_index("c")
    base = wid * ROWS_PER_W

    d_bufs = (d_v0, d_v1)
    j_bufs = (j_v0, j_v1)
    lh_bufs = (lh_v0, lh_v1)
    lp_bufs = (lp_v0, lp_v1)

    pltpu.sync_copy(qv_hbm, qv_v)

    def in_copy(c, buf):
        return pltpu.make_async_copy(
            deg_hbm.at[pl.ds(base + c * CHUNK, CHUNK)], d_bufs[buf], sem_in.at[buf])

    def out_copy(c, buf):
        sl = pl.ds(base + c * CHUNK, CHUNK)
        return (pltpu.make_async_copy(j_bufs[buf], j_hbm.at[sl], sem_out.at[buf]),
                pltpu.make_async_copy(lh_bufs[buf], lh_hbm.at[sl], sem_out.at[buf]),
                pltpu.make_async_copy(lp_bufs[buf], lp_hbm.at[sl], sem_out.at[buf]))

    in_copy(0, 0).start()

    i31 = jnp.full((L,), K - 1, jnp.int32)

    for c in range(NCHUNK):
        buf = c % 2
        if c + 1 < NCHUNK:
            in_copy(c + 1, 1 - buf).start()
        in_copy(c, buf).wait()

        qmax = plsc.load_gather(qv_v, [i31])
        d_v = d_bufs[buf]
        j_v = j_bufs[buf]
        lh_v = lh_bufs[buf]
        lp_v = lp_bufs[buf]

        def step(i, _):
            d = d_v[pl.ds(i * L, L)]
            j = jnp.zeros((L,), jnp.int32)
            for stepw in (16, 8, 4, 2, 1):
                cand = j + stepw
                v = plsc.load_gather(qv_v, [jnp.minimum(cand, K - 1)])
                j = jnp.where((cand <= K - 1) & (d >= v), cand, j)
            lower = plsc.load_gather(qv_v, [j])
            upper = plsc.load_gather(qv_v, [jnp.minimum(j + 1, K - 1)])
            pos = (d - lower) / (upper - lower + 1e-10)
            pos = jnp.clip(pos, 0.0, 1.0)
            m = (d >= lower) & (d < upper)
            over = d >= qmax
            jenc = jnp.where(over, K, jnp.where(m, j, -1)).astype(jnp.float32)
            sl = pl.ds(i * L, L)
            j_v[sl] = jenc
            lh_v[sl] = _fast_log(1.0 - pos + 1e-30)
            lp_v[sl] = _fast_log(pos + 1e-30)
            return 0

        lax.fori_loop(0, CHUNK // L, step, 0, unroll=2)
        for cp in out_copy(c, buf):
            cp.start()

    for cc in range(NCHUNK):
        for cp in out_copy(cc, cc % 2):
            cp.wait()


def _expand_store(jbr, lhr, lpr, o_ref):
    gsel = lax.broadcasted_iota(jnp.int32, (GEXP, GEXP * K), 1) // K
    grow = lax.broadcasted_iota(jnp.int32, (GEXP, GEXP * K), 0)
    ee = (gsel == grow).astype(jnp.float32)
    dn = (((0,), (0,)), ((), ()))
    jb = lax.dot_general(jbr, ee, dn, preferred_element_type=jnp.float32)
    lb = lax.dot_general(lhr, ee, dn, preferred_element_type=jnp.float32)
    pb = lax.dot_general(lpr, ee, dn, preferred_element_type=jnp.float32)
    col = (lax.broadcasted_iota(jnp.int32, (TCB, GEXP * K), 1) %
           K).astype(jnp.float32)
    out = jnp.where(col == jb, lb, LOG_EPS)
    v31 = jnp.where(jb == float(K), 0.0,
                    jnp.where(jb == float(K - 2), pb, LOG_EPS))
    out = jnp.where(col == float(K - 1), v31, out)
    for g in range(GEXP):
        b_off = (g * TCB) // S
        s_off = (g * TCB) % S
        o_ref[b_off, s_off:s_off + TCB, :] = lax.slice(
            out, (0, g * K), (TCB, g * K + K))


def _tc_compute_body(d_ref, qv_ref, qvn_ref, o_ref):
    d2 = d_ref[...]
    j = jnp.zeros(d2.shape, jnp.int32)
    lower = jnp.full(d2.shape, qv_ref[0], jnp.float32)
    upper = jnp.full(d2.shape, qvn_ref[0], jnp.float32)
    for c in range(K):
        gec = d2 >= qv_ref[c]
        j = j + gec.astype(jnp.int32)
        lower = jnp.where(gec, qv_ref[c], lower)
        upper = jnp.where(gec, qvn_ref[c], upper)
    pos = (d2 - lower) / (upper - lower + 1e-10)
    pos = jnp.clip(pos, 0.0, 1.0)
    m = (d2 >= lower) & (d2 < upper)
    over = d2 >= qv_ref[K - 1]
    jenc = jnp.where(over, K, jnp.where(m, j - 1, -1)).astype(jnp.float32)
    lh = _fast_log(1.0 - pos + 1e-30)
    lp = _fast_log(pos + 1e-30)
    _expand_store(jenc, lh, lp, o_ref)


def _tc_expand_body(j_ref, lh_ref, lp_ref, prev_ref, o_ref):
    del prev_ref
    _expand_store(j_ref[...], lh_ref[...], lp_ref[...], o_ref)


@jax.jit
def kernel(degrees, quantile_values):
    qv = quantile_values
    qvn = jnp.concatenate([qv[1:], qv[K - 1:]])
    deg_sc = degrees.reshape(R)[:R_SC]

    mesh = plsc.VectorSubcoreMesh(
        core_axis_name="c", subcore_axis_name="s", num_cores=NC, num_subcores=NS)
    j_arr, lh_arr, lp_arr = pl.kernel(
        _sc_body,
        out_type=(jax.ShapeDtypeStruct((R_SC,), jnp.float32),
                  jax.ShapeDtypeStruct((R_SC,), jnp.float32),
                  jax.ShapeDtypeStruct((R_SC,), jnp.float32)),
        mesh=mesh,
        compiler_params=pltpu.CompilerParams(needs_layout_passes=False),
        scratch_types=[
            pltpu.VMEM((K,), jnp.float32),
            pltpu.VMEM((CHUNK,), jnp.float32),
            pltpu.VMEM((CHUNK,), jnp.float32),
            pltpu.VMEM((CHUNK,), jnp.float32),
            pltpu.VMEM((CHUNK,), jnp.float32),
            pltpu.VMEM((CHUNK,), jnp.float32),
            pltpu.VMEM((CHUNK,), jnp.float32),
            pltpu.VMEM((CHUNK,), jnp.float32),
            pltpu.VMEM((CHUNK,), jnp.float32),
            pltpu.SemaphoreType.DMA((2,)),
            pltpu.SemaphoreType.DMA((2,)),
        ],
    )(deg_sc, qv)

    deg2 = degrees.reshape(R // TCB, TCB)
    out1 = pl.pallas_call(
        _tc_compute_body,
        grid=((B - SPLIT) // 2,),
        in_specs=[
            pl.BlockSpec((GEXP, TCB), lambda i: (SPLIT // 2 + i, 0)),
            pl.BlockSpec(memory_space=pltpu.MemorySpace.SMEM),
            pl.BlockSpec(memory_space=pltpu.MemorySpace.SMEM),
        ],
        out_specs=pl.BlockSpec((2, S, K), lambda i: (SPLIT // 2 + i, 0, 0)),
        out_shape=jax.ShapeDtypeStruct((B, S, K), jnp.float32),
    )(deg2, qv, qvn)

    j2 = j_arr.reshape(R_SC // TCB, TCB)
    lh2 = lh_arr.reshape(R_SC // TCB, TCB)
    lp2 = lp_arr.reshape(R_SC // TCB, TCB)
    out = pl.pallas_call(
        _tc_expand_body,
        grid=(SPLIT // 2,),
        in_specs=[
            pl.BlockSpec((GEXP, TCB), lambda i: (i, 0)),
            pl.BlockSpec((GEXP, TCB), lambda i: (i, 0)),
            pl.BlockSpec((GEXP, TCB), lambda i: (i, 0)),
            pl.BlockSpec(memory_space=pltpu.MemorySpace.HBM),
        ],
        out_specs=pl.BlockSpec((2, S, K), lambda i: (i, 0, 0)),
        out_shape=jax.ShapeDtypeStruct((B, S, K), jnp.float32),
        input_output_aliases={3: 0},
    )(j2, lh2, lp2, out1)
    return out


# speedup vs baseline: 1.1102x; 1.0019x over previous
import math

import jax
import jax.numpy as jnp
from jax import lax
from jax.experimental import pallas as pl
from jax.experimental.pallas import tpu as pltpu
from jax.experimental.pallas import tpu_sc as plsc

NC = 2
NS = 16
NW = NC * NS
L = 16

B, S, K = 16, 8192, 32
R = B * S
SPLIT = 2
R_SC = SPLIT * S
ROWS_PER_W = R_SC // NW
NCHUNK = 2
CHUNK = ROWS_PER_W // NCHUNK
LOG_EPS = float(math.log(1e-30))
LN2 = 0.6931471805599453
SQRT2 = 1.4142135623730951
TCB = 128
GEXP = 128
SBLK = 8192


def _fast_log(x):
    bits = lax.bitcast_convert_type(x, jnp.int32)
    e = lax.shift_right_logical(bits, 23) - 127
    m = lax.bitcast_convert_type(
        jnp.bitwise_or(jnp.bitwise_and(bits, 0x7FFFFF), 0x3F800000), jnp.float32)
    big = m >= SQRT2
    m = jnp.where(big, m * 0.5, m)
    e = jnp.where(big, e + 1, e).astype(jnp.float32)
    s = (m - 1.0) / (m + 1.0)
    z = s * s
    poly = 1.0 + z * (1.0 / 3.0 + z * (1.0 / 5.0 + z * (1.0 / 7.0 + z * (1.0 / 9.0))))
    return e * LN2 + 2.0 * s * poly


def _sc_body(deg_hbm, qv_hbm, j_hbm, lh_hbm, lp_hbm, qv_v,
             d_v0, d_v1, j_v0, j_v1, lh_v0, lh_v1, lp_v0, lp_v1,
             sem_in, sem_out):
    wid = lax.axis_index("s") * NC + lax.axis_index("c")
    base = wid * ROWS_PER_W

    d_bufs = (d_v0, d_v1)
    j_bufs = (j_v0, j_v1)
    lh_bufs = (lh_v0, lh_v1)
    lp_bufs = (lp_v0, lp_v1)

    pltpu.sync_copy(qv_hbm, qv_v)

    def in_copy(c, buf):
        return pltpu.make_async_copy(
            deg_hbm.at[pl.ds(base + c * CHUNK, CHUNK)], d_bufs[buf], sem_in.at[buf])

    def out_copy(c, buf):
        sl = pl.ds(base + c * CHUNK, CHUNK)
        return (pltpu.make_async_copy(j_bufs[buf], j_hbm.at[sl], sem_out.at[buf]),
                pltpu.make_async_copy(lh_bufs[buf], lh_hbm.at[sl], sem_out.at[buf]),
                pltpu.make_async_copy(lp_bufs[buf], lp_hbm.at[sl], sem_out.at[buf]))

    in_copy(0, 0).start()

    i31 = jnp.full((L,), K - 1, jnp.int32)

    for c in range(NCHUNK):
        buf = c % 2
        if c + 1 < NCHUNK:
            in_copy(c + 1, 1 - buf).start()
        in_copy(c, buf).wait()

        qmax = plsc.load_gather(qv_v, [i31])
        d_v = d_bufs[buf]
        j_v = j_bufs[buf]
        lh_v = lh_bufs[buf]
        lp_v = lp_bufs[buf]

        def step(i, _):
            d = d_v[pl.ds(i * L, L)]
            j = jnp.zeros((L,), jnp.int32)
            for stepw in (16, 8, 4, 2, 1):
                cand = j + stepw
                v = plsc.load_gather(qv_v, [jnp.minimum(cand, K - 1)])
                j = jnp.where((cand <= K - 1) & (d >= v), cand, j)
            lower = plsc.load_gather(qv_v, [j])
            upper = plsc.load_gather(qv_v, [jnp.minimum(j + 1, K - 1)])
            pos = (d - lower) / (upper - lower + 1e-10)
            pos = jnp.clip(pos, 0.0, 1.0)
            m = (d >= lower) & (d < upper)
            over = d >= qmax
            jenc = jnp.where(over, K, jnp.where(m, j, -1)).astype(jnp.float32)
            sl = pl.ds(i * L, L)
            j_v[sl] = jenc
            lh_v[sl] = _fast_log(1.0 - pos + 1e-30)
            lp_v[sl] = _fast_log(pos + 1e-30)
            return 0

        lax.fori_loop(0, CHUNK // L, step, 0, unroll=2)
        for cp in out_copy(c, buf):
            cp.start()

    for cc in range(NCHUNK):
        for cp in out_copy(cc, cc % 2):
            cp.wait()


def _expand_store(jbr, lhr, lpr, o_ref):
    gsel = lax.broadcasted_iota(jnp.int32, (GEXP, GEXP * K), 1) // K
    grow = lax.broadcasted_iota(jnp.int32, (GEXP, GEXP * K), 0)
    ee = (gsel == grow).astype(jnp.float32)
    dn = (((0,), (0,)), ((), ()))
    jb = lax.dot_general(jbr, ee, dn, preferred_element_type=jnp.float32)
    lb = lax.dot_general(lhr, ee, dn, preferred_element_type=jnp.float32)
    pb = lax.dot_general(lpr, ee, dn, preferred_element_type=jnp.float32)
    col = (lax.broadcasted_iota(jnp.int32, (TCB, GEXP * K), 1) %
           K).astype(jnp.float32)
    out = jnp.where(col == jb, lb, LOG_EPS)
    v31 = jnp.where(jb == float(K), 0.0,
                    jnp.where(jb == float(K - 2), pb, LOG_EPS))
    out = jnp.where(col == float(K - 1), v31, out)
    for g in range(GEXP):
        b_off = (g * TCB) // S
        s_off = (g * TCB) % S
        o_ref[b_off, s_off:s_off + TCB, :] = lax.slice(
            out, (0, g * K), (TCB, g * K + K))


def _tc_compute_body(d_ref, qv_ref, qvn_ref, o_ref):
    d2 = d_ref[...]
    j = jnp.zeros(d2.shape, jnp.int32)
    lower = jnp.full(d2.shape, qv_ref[0], jnp.float32)
    upper = jnp.full(d2.shape, qvn_ref[0], jnp.float32)
    for c in range(K):
        gec = d2 >= qv_ref[c]
        j = j + gec.astype(jnp.int32)
        lower = jnp.where(gec, qv_ref[c], lower)
        upper = jnp.where(gec, qvn_ref[c], upper)
    pos = (d2 - lower) / (upper - lower + 1e-10)
    pos = jnp.clip(pos, 0.0, 1.0)
    m = (d2 >= lower) & (d2 < upper)
    over = d2 >= qv_ref[K - 1]
    jenc = jnp.where(over, K, jnp.where(m, j - 1, -1)).astype(jnp.float32)
    lh = _fast_log(1.0 - pos + 1e-30)
    lp = _fast_log(pos + 1e-30)
    _expand_store(jenc, lh, lp, o_ref)


def _tc_expand_body(j_ref, lh_ref, lp_ref, prev_ref, o_ref):
    del prev_ref
    _expand_store(j_ref[...], lh_ref[...], lp_ref[...], o_ref)


@jax.jit
def kernel(degrees, quantile_values):
    qv = quantile_values
    qvn = jnp.concatenate([qv[1:], qv[K - 1:]])
    deg_sc = degrees.reshape(R)[:R_SC]

    mesh = plsc.VectorSubcoreMesh(
        core_axis_name="c", subcore_axis_name="s", num_cores=NC, num_subcores=NS)
    j_arr, lh_arr, lp_arr = pl.kernel(
        _sc_body,
        out_type=(jax.ShapeDtypeStruct((R_SC,), jnp.float32),
                  jax.ShapeDtypeStruct((R_SC,), jnp.float32),
                  jax.ShapeDtypeStruct((R_SC,), jnp.float32)),
        mesh=mesh,
        compiler_params=pltpu.CompilerParams(needs_layout_passes=False),
        scratch_types=[
            pltpu.VMEM((K,), jnp.float32),
            pltpu.VMEM((CHUNK,), jnp.float32),
            pltpu.VMEM((CHUNK,), jnp.float32),
            pltpu.VMEM((CHUNK,), jnp.float32),
            pltpu.VMEM((CHUNK,), jnp.float32),
            pltpu.VMEM((CHUNK,), jnp.float32),
            pltpu.VMEM((CHUNK,), jnp.float32),
            pltpu.VMEM((CHUNK,), jnp.float32),
            pltpu.VMEM((CHUNK,), jnp.float32),
            pltpu.SemaphoreType.DMA((2,)),
            pltpu.SemaphoreType.DMA((2,)),
        ],
    )(deg_sc, qv)

    deg2 = degrees.reshape(R // TCB, TCB)
    out1 = pl.pallas_call(
        _tc_compute_body,
        grid=((B - SPLIT) // 2,),
        in_specs=[
            pl.BlockSpec((GEXP, TCB), lambda i: (SPLIT // 2 + i, 0)),
            pl.BlockSpec(memory_space=pltpu.MemorySpace.SMEM),
            pl.BlockSpec(memory_space=pltpu.MemorySpace.SMEM),
        ],
        out_specs=pl.BlockSpec((2, S, K), lambda i: (SPLIT // 2 + i, 0, 0)),
        out_shape=jax.ShapeDtypeStruct((B, S, K), jnp.float32),
    )(deg2, qv, qvn)

    j2 = j_arr.reshape(R_SC // TCB, TCB)
    lh2 = lh_arr.reshape(R_SC // TCB, TCB)
    lp2 = lp_arr.reshape(R_SC // TCB, TCB)
    out = pl.pallas_call(
        _tc_expand_body,
        grid=(SPLIT // 2,),
        in_specs=[
            pl.BlockSpec((GEXP, TCB), lambda i: (i, 0)),
            pl.BlockSpec((GEXP, TCB), lambda i: (i, 0)),
            pl.BlockSpec((GEXP, TCB), lambda i: (i, 0)),
            pl.BlockSpec(memory_space=pltpu.MemorySpace.HBM),
        ],
        out_specs=pl.BlockSpec((2, S, K), lambda i: (i, 0, 0)),
        out_shape=jax.ShapeDtypeStruct((B, S, K), jnp.float32),
        input_output_aliases={3: 0},
    )(j2, lh2, lp2, out1)
    return out
